# baseline (speedup 1.0000x reference)
import functools

import jax
import jax.numpy as jnp
from jax import lax
from jax.experimental import pallas as pl
from jax.experimental.pallas import tpu as pltpu
from jax.experimental.pallas import tpu_sc as plsc

B = 4096
L = 50
D = 64
NR = 10

_info = plsc.get_sparse_core_info()
NC = _info.num_cores
NS = _info.num_subcores
LANES = _info.num_lanes
NW = NC * NS
UPW = B // NW
UC = 8
ROWS = UC * L
NCH = UPW // UC
GSUB = 80
NG = ROWS // GSUB

_mesh = plsc.VectorSubcoreMesh(core_axis_name="c", subcore_axis_name="s")
_params = pltpu.CompilerParams(
    needs_layout_passes=False, use_tc_tiling_on_sc=False)


@functools.partial(
    pl.kernel,
    mesh=_mesh,
    compiler_params=_params,
    out_type=jax.ShapeDtypeStruct((B, D), jnp.float32),
    scratch_types=[
        pltpu.VMEM((UPW, L), jnp.int32),
        pltpu.VMEM((UPW * L,), jnp.int32),
        pltpu.VMEM((UPW, L), jnp.float32),
        pltpu.VMEM((UPW, 4 * LANES), jnp.int32),
        pltpu.VMEM((NR, D), jnp.float32),
        pltpu.VMEM((2, ROWS, D), jnp.float32),
        pltpu.VMEM((UC, D), jnp.float32),
        pltpu.SemaphoreType.DMA,
        pltpu.SemaphoreType.DMA,
    ],
)
def _user_kernel(fid_hbm, rat_hbm, ftab_hbm, rt_hbm, out_hbm,
                 fid2_v, fid_v, rat_v, ridx_v, rt_v, rows_v, acc_v, sem0,
                 sem1):
  wid = lax.axis_index("s") * NC + lax.axis_index("c")
  ubase = pl.multiple_of(wid * UPW, UPW)

  pltpu.sync_copy(fid_hbm.at[pl.ds(ubase, UPW)], fid2_v)
  pltpu.sync_copy(rat_hbm.at[pl.ds(ubase, UPW)], rat_v)
  pltpu.sync_copy(rt_hbm, rt_v)

  lane = lax.iota(jnp.int32, LANES)
  def _flat(i, carry):
    p = i * LANES + lane
    v = plsc.load_gather(fid2_v, [p // L, p % L])
    fid_v[pl.ds(i * LANES, LANES)] = v
    return carry
  lax.fori_loop(0, UPW * L // LANES, _flat, 0, unroll=4)

  def _ridx(u, carry):
    uvec = jnp.full((LANES,), u, jnp.int32)
    for g in range(4):
      lcl = jnp.minimum(g * LANES + lane, L - 1)
      r = plsc.load_gather(rat_v, [uvec, lcl])
      ridx_v[u, pl.ds(g * LANES, LANES)] = jnp.clip(
          ((r - 0.5) * 2.0).astype(jnp.int32), 0, 9)
    return carry
  lax.fori_loop(0, UPW, _ridx, 0, unroll=2)

  sems = (sem0, sem1)

  def _start(j, buf, sem):
    crow = pl.multiple_of(j * ROWS, 8)
    for k in range(NG):
      pltpu.async_copy(
          ftab_hbm.at[fid_v.at[pl.ds(crow + k * GSUB, GSUB)]],
          rows_v.at[buf, pl.ds(k * GSUB, GSUB)], sem)

  def _drain(buf, sem):
    for k in range(NG):
      pltpu.make_async_copy(
          ftab_hbm.at[fid_v.at[pl.ds(k * GSUB, GSUB)]],
          rows_v.at[buf, pl.ds(k * GSUB, GSUB)], sem).wait()

  zeros = jnp.zeros((LANES,), jnp.float32)
  _start(0, 0, sem0)

  def _compute(c, buf):
    cu = c * UC
    def _user(u, carry):
      accs = [zeros, zeros, zeros, zeros]
      rbase = u * L
      for g in range(4):
        rv = ridx_v[cu + u, pl.ds(g * LANES, LANES)]
        for j in range(LANES if g < 3 else L - 3 * LANES):
          ridx = rv[j]
          row = rbase + g * LANES + j
          for dg in range(4):
            rtv = rt_v[ridx, pl.ds(dg * LANES, LANES)]
            fv = rows_v[buf, row, pl.ds(dg * LANES, LANES)]
            accs[dg] = accs[dg] + fv * rtv
      for dg in range(4):
        acc_v[u, pl.ds(dg * LANES, LANES)] = accs[dg]
      return carry
    lax.fori_loop(0, UC, _user, 0)
    pltpu.sync_copy(acc_v, out_hbm.at[pl.ds(ubase + cu, UC)])

  def _pair(c2, carry):
    c = c2 * 2
    for par in range(2):
      @pl.when(c + par + 1 < NCH)
      def _():
        _start(c + par + 1, 1 - par, sems[1 - par])
      _drain(par, sems[par])
      _compute(c + par, par)
    return carry
  lax.fori_loop(0, NCH // 2, _pair, 0)


@functools.partial(
    pl.kernel,
    mesh=_mesh,
    compiler_params=_params,
    out_type=jax.ShapeDtypeStruct((B, D), jnp.float32),
    scratch_types=[
        pltpu.VMEM((UPW,), jnp.int32),
        pltpu.VMEM((UPW, D), jnp.float32),
        pltpu.SemaphoreType.DMA,
    ],
)
def _item_kernel(iid_hbm, itab_hbm, out_hbm, iid_v, item_rows, sem):
  wid = lax.axis_index("s") * NC + lax.axis_index("c")
  ubase = pl.multiple_of(wid * UPW, UPW)
  pltpu.sync_copy(iid_hbm.at[pl.ds(ubase, UPW)], iid_v)
  pltpu.async_copy(itab_hbm.at[iid_v], item_rows, sem).wait()
  pltpu.sync_copy(item_rows, out_hbm.at[pl.ds(ubase, UPW)])


def kernel(feature_ids, feature_ratings, item_ids, feature_table,
           rating_table, item_table):
  fid = feature_ids.astype(jnp.int32)
  iid = item_ids.astype(jnp.int32)
  user = _user_kernel(fid, feature_ratings, feature_table, rating_table)
  item = _item_kernel(iid, item_table)
  return jnp.stack((user, item), axis=1)
